# baseline (speedup 1.0000x reference)
import functools

import jax
import jax.numpy as jnp
from jax import lax
from jax.experimental import pallas as pl
from jax.experimental.pallas import tpu as pltpu
from jax.experimental.pallas import tpu_sc as plsc

B = 16384
D = 32
NC, NS = 2, 16
NW = NC * NS
BPW = B // NW
CHUNK = 128
NCH = BPW // CHUNK


def _sc_gather(u2, i2, user_gmf, item_gmf, user_mlp, item_mlp):
    mesh = plsc.VectorSubcoreMesh(core_axis_name="c", subcore_axis_name="s")
    row = jax.ShapeDtypeStruct((B, D), jnp.float32)

    @functools.partial(
        pl.kernel,
        mesh=mesh,
        out_type=(row, row, row, row),
        compiler_params=pltpu.CompilerParams(use_tc_tiling_on_sc=False),
        scratch_types=[
            pltpu.VMEM((NCH, CHUNK), jnp.int32),
            pltpu.VMEM((NCH, CHUNK), jnp.int32),
            pltpu.VMEM((BPW, D), jnp.float32),
            pltpu.VMEM((BPW, D), jnp.float32),
            pltpu.VMEM((BPW, D), jnp.float32),
            pltpu.VMEM((BPW, D), jnp.float32),
            pltpu.SemaphoreType.DMA,
        ],
    )
    def k(u_hbm, i_hbm, ug_t, ig_t, um_t, im_t,
          ug_o, ig_o, um_o, im_o,
          u_v, i_v, bug, big, bum, bim, sem):
        wid = lax.axis_index("s") * NC + lax.axis_index("c")
        pltpu.sync_copy(u_hbm.at[pl.ds(wid * NCH, NCH)], u_v)
        pltpu.sync_copy(i_hbm.at[pl.ds(wid * NCH, NCH)], i_v)
        cps = []
        for j in range(NCH):
            sl = pl.ds(j * CHUNK, CHUNK)
            cps.append(pltpu.async_copy(ug_t.at[u_v.at[j]], bug.at[sl], sem))
            cps.append(pltpu.async_copy(ig_t.at[i_v.at[j]], big.at[sl], sem))
            cps.append(pltpu.async_copy(um_t.at[u_v.at[j]], bum.at[sl], sem))
            cps.append(pltpu.async_copy(im_t.at[i_v.at[j]], bim.at[sl], sem))
        for c in cps:
            c.wait()
        base = wid * BPW
        pltpu.sync_copy(bug, ug_o.at[pl.ds(base, BPW)])
        pltpu.sync_copy(big, ig_o.at[pl.ds(base, BPW)])
        pltpu.sync_copy(bum, um_o.at[pl.ds(base, BPW)])
        pltpu.sync_copy(bim, im_o.at[pl.ds(base, BPW)])

    return k(u2, i2, user_gmf, item_gmf, user_mlp, item_mlp)


BLK = 2048


def _tc_mlp(ug, ig, um, im, W1, b1, W2, b2, W3, b3, wg, wx, bout):
    def body(ug_ref, ig_ref, um_ref, im_ref, w1_ref, b1_ref, w2_ref, b2_ref,
             w3_ref, b3_ref, wg_ref, wx_ref, bo_ref, o_ref):
        gmf = ug_ref[...] * ig_ref[...]
        x = jnp.concatenate([um_ref[...], im_ref[...]], axis=-1)
        x = jnp.maximum(
            jnp.dot(x, w1_ref[...], preferred_element_type=jnp.float32)
            + b1_ref[...], 0.0)
        x = jnp.maximum(
            jnp.dot(x, w2_ref[...], preferred_element_type=jnp.float32)
            + b2_ref[...], 0.0)
        x = jnp.maximum(
            jnp.dot(x, w3_ref[...], preferred_element_type=jnp.float32)
            + b3_ref[...], 0.0)
        o = (jnp.sum(gmf * wg_ref[...], axis=-1)
             + jnp.sum(x * wx_ref[...], axis=-1) + bo_ref[0, 0])
        o_ref[...] = 1.0 / (1.0 + jnp.exp(-o))

    blk_spec = pl.BlockSpec((BLK, D), lambda n: (n, 0))
    full = lambda s: pl.BlockSpec(s, lambda n: tuple(0 for _ in s))
    return pl.pallas_call(
        body,
        grid=(B // BLK,),
        in_specs=[
            blk_spec, blk_spec, blk_spec, blk_spec,
            full((64, 128)), full((1, 128)),
            full((128, 64)), full((1, 64)),
            full((64, 32)), full((1, 32)),
            full((1, 32)), full((1, 32)), full((1, 1)),
        ],
        out_specs=pl.BlockSpec((BLK,), lambda n: (n,)),
        out_shape=jax.ShapeDtypeStruct((B,), jnp.float32),
    )(ug, ig, um, im, W1, b1, W2, b2, W3, b3, wg, wx, bout)


def kernel(u, i, user_gmf, item_gmf, user_mlp, item_mlp,
           W1, b1, W2, b2, W3, b3, Wout, bout):
    u2 = u.reshape(NW * NCH, CHUNK)
    i2 = i.reshape(NW * NCH, CHUNK)
    ug, ig, um, im = _sc_gather(u2, i2, user_gmf, item_gmf, user_mlp, item_mlp)
    wg = Wout[:D, 0].reshape(1, D)
    wx = Wout[D:, 0].reshape(1, D)
    return _tc_mlp(ug, ig, um, im,
                   W1, b1.reshape(1, -1), W2, b2.reshape(1, -1),
                   W3, b3.reshape(1, -1), wg, wx, bout.reshape(1, 1))


# speedup vs baseline: 3.3451x; 3.3451x over previous
import functools

import jax
import jax.numpy as jnp
from jax import lax
from jax.experimental import pallas as pl
from jax.experimental.pallas import tpu as pltpu
from jax.experimental.pallas import tpu_sc as plsc

B = 16384
D = 32
NC, NS = 2, 16
NW = NC * NS
BPW = B // NW
GROUP = 16


RING = 3


def _sc_gather(u, i, ugT, igT, umT, imT):
    mesh = plsc.VectorSubcoreMesh(core_axis_name="c", subcore_axis_name="s")
    row = jax.ShapeDtypeStruct((D, B), jnp.float32)

    @functools.partial(
        pl.kernel,
        mesh=mesh,
        out_type=(row, row, row, row),
        compiler_params=pltpu.CompilerParams(needs_layout_passes=False),
        scratch_types=[
            pltpu.VMEM((BPW,), jnp.int32),
            pltpu.VMEM((BPW,), jnp.int32),
            pltpu.VMEM((D, BPW), jnp.float32),
            pltpu.VMEM((D, BPW), jnp.float32),
            pltpu.VMEM((D, BPW), jnp.float32),
            pltpu.VMEM((D, BPW), jnp.float32),
        ]
        + [pltpu.VMEM((D, 128), jnp.float32)] * (4 * RING)
        + [pltpu.SemaphoreType.DMA] * RING,
    )
    def k(u_hbm, i_hbm, ug_t, ig_t, um_t, im_t,
          ug_o, ig_o, um_o, im_o,
          u_v, i_v, bug, big, bum, bim, *rest):
        rings = rest[:4 * RING]
        sems = rest[4 * RING:]
        outs = (bug, bum, big, bim)
        wid = lax.axis_index("s") * NC + lax.axis_index("c")
        base = wid * BPW
        pltpu.sync_copy(u_hbm.at[pl.ds(base, BPW)], u_v)
        pltpu.sync_copy(i_hbm.at[pl.ds(base, BPW)], i_v)
        rows_lo = lax.iota(jnp.int32, GROUP)
        rows_hi = rows_lo + GROUP

        def step(g, _):
            uvec = u_v[pl.ds(g * GROUP, GROUP)]
            ivec = i_v[pl.ds(g * GROUP, GROUP)]
            us = [uvec[k_] for k_ in range(GROUP)]
            is_ = [ivec[k_] for k_ in range(GROUP)]

            def fire_one(k_):
                s = k_ % RING
                utc = pl.multiple_of((us[k_] >> 7) << 7, 128)
                itc = pl.multiple_of((is_[k_] >> 7) << 7, 128)
                return [
                    pltpu.async_copy(ug_t.at[:, pl.ds(utc, 128)],
                                     rings[0 * RING + s], sems[s]),
                    pltpu.async_copy(um_t.at[:, pl.ds(utc, 128)],
                                     rings[1 * RING + s], sems[s]),
                    pltpu.async_copy(ig_t.at[:, pl.ds(itc, 128)],
                                     rings[2 * RING + s], sems[s]),
                    pltpu.async_copy(im_t.at[:, pl.ds(itc, 128)],
                                     rings[3 * RING + s], sems[s]),
                ]

            def extract_one(k_, cps):
                s = k_ % RING
                for c in cps:
                    c.wait()
                slot = g * GROUP + k_
                cols_s = jnp.full((GROUP,), slot, jnp.int32)
                uc = jnp.full((GROUP,), us[k_] & 127, jnp.int32)
                ic = jnp.full((GROUP,), is_[k_] & 127, jnp.int32)
                for t, (out, cvec) in enumerate(
                        zip(outs, (uc, uc, ic, ic))):
                    buf = rings[t * RING + s]
                    v_lo = plsc.load_gather(buf, [rows_lo, cvec])
                    v_hi = plsc.load_gather(buf, [rows_hi, cvec])
                    plsc.store_scatter(out, [rows_lo, cols_s], v_lo)
                    plsc.store_scatter(out, [rows_hi, cols_s], v_hi)

            pending = {}
            for k_ in range(RING - 1):
                pending[k_] = fire_one(k_)
            for k_ in range(GROUP):
                if k_ + RING - 1 < GROUP:
                    pending[k_ + RING - 1] = fire_one(k_ + RING - 1)
                extract_one(k_, pending.pop(k_))
            return 0

        lax.fori_loop(0, BPW // GROUP, step, 0)
        dst = pl.ds(base, BPW)
        pltpu.sync_copy(bug, ug_o.at[:, dst])
        pltpu.sync_copy(bum, um_o.at[:, dst])
        pltpu.sync_copy(big, ig_o.at[:, dst])
        pltpu.sync_copy(bim, im_o.at[:, dst])

    return k(u, i, ugT, igT, umT, imT)


BLK = 2048


def _tc_mlp(ugT, igT, umT, imT, w1t, b1, w2t, b2, w3t, b3, wg, wx, bout):
    def body(ug_ref, ig_ref, um_ref, im_ref, w1_ref, b1_ref, w2_ref, b2_ref,
             w3_ref, b3_ref, wg_ref, wx_ref, bo_ref, o_ref):
        gmf = ug_ref[...] * ig_ref[...]
        x = jnp.concatenate([um_ref[...], im_ref[...]], axis=0)
        x = jnp.maximum(
            jnp.dot(w1_ref[...], x, preferred_element_type=jnp.float32)
            + b1_ref[...], 0.0)
        x = jnp.maximum(
            jnp.dot(w2_ref[...], x, preferred_element_type=jnp.float32)
            + b2_ref[...], 0.0)
        x = jnp.maximum(
            jnp.dot(w3_ref[...], x, preferred_element_type=jnp.float32)
            + b3_ref[...], 0.0)
        o = (jnp.dot(wg_ref[...], gmf, preferred_element_type=jnp.float32)
             + jnp.dot(wx_ref[...], x, preferred_element_type=jnp.float32)
             + bo_ref[...])
        o_ref[...] = 1.0 / (1.0 + jnp.exp(-o))

    blk_spec = pl.BlockSpec((D, BLK), lambda n: (0, n))
    full = lambda s: pl.BlockSpec(s, lambda n: tuple(0 for _ in s))
    return pl.pallas_call(
        body,
        grid=(B // BLK,),
        in_specs=[
            blk_spec, blk_spec, blk_spec, blk_spec,
            full((128, 64)), full((128, 1)),
            full((64, 128)), full((64, 1)),
            full((32, 64)), full((32, 1)),
            full((1, 32)), full((1, 32)), full((1, 1)),
        ],
        out_specs=pl.BlockSpec((1, BLK), lambda n: (0, n)),
        out_shape=jax.ShapeDtypeStruct((1, B), jnp.float32),
    )(ugT, igT, umT, imT, w1t, b1, w2t, b2, w3t, b3, wg, wx, bout)


def kernel(u, i, user_gmf, item_gmf, user_mlp, item_mlp,
           W1, b1, W2, b2, W3, b3, Wout, bout):
    ugT, igT = jnp.transpose(user_gmf), jnp.transpose(item_gmf)
    umT, imT = jnp.transpose(user_mlp), jnp.transpose(item_mlp)
    g_ugT, g_igT, g_umT, g_imT = _sc_gather(u, i, ugT, igT, umT, imT)
    out = _tc_mlp(g_ugT, g_igT, g_umT, g_imT,
                  W1.T, b1.reshape(-1, 1), W2.T, b2.reshape(-1, 1),
                  W3.T, b3.reshape(-1, 1),
                  Wout[:D, 0].reshape(1, D), Wout[D:, 0].reshape(1, D),
                  bout.reshape(1, 1))
    return out.reshape(B)


# speedup vs baseline: 3.3752x; 1.0090x over previous
import functools

import jax
import jax.numpy as jnp
from jax import lax
from jax.experimental import pallas as pl
from jax.experimental.pallas import tpu as pltpu
from jax.experimental.pallas import tpu_sc as plsc

B = 16384
D = 32
NC, NS = 2, 16
NW = NC * NS
BPW = B // NW
GROUP = 16


RING = 3


def _sc_gather(u, i, ugT, igT, umT, imT):
    mesh = plsc.VectorSubcoreMesh(core_axis_name="c", subcore_axis_name="s")
    row = jax.ShapeDtypeStruct((D, B), jnp.float32)

    @functools.partial(
        pl.kernel,
        mesh=mesh,
        out_type=(row, row, row, row),
        compiler_params=pltpu.CompilerParams(needs_layout_passes=False),
        scratch_types=[
            pltpu.VMEM((BPW,), jnp.int32),
            pltpu.VMEM((BPW,), jnp.int32),
            pltpu.VMEM((D, BPW), jnp.float32),
            pltpu.VMEM((D, BPW), jnp.float32),
            pltpu.VMEM((D, BPW), jnp.float32),
            pltpu.VMEM((D, BPW), jnp.float32),
        ]
        + [pltpu.VMEM((D, 128), jnp.float32)] * (4 * RING)
        + [pltpu.SemaphoreType.DMA] * RING,
    )
    def k(u_hbm, i_hbm, ug_t, ig_t, um_t, im_t,
          ug_o, ig_o, um_o, im_o,
          u_v, i_v, bug, big, bum, bim, *rest):
        rings = rest[:4 * RING]
        sems = rest[4 * RING:]
        outs = (bug, bum, big, bim)
        wid = lax.axis_index("s") * NC + lax.axis_index("c")
        base = wid * BPW
        pltpu.sync_copy(u_hbm.at[pl.ds(base, BPW)], u_v)
        pltpu.sync_copy(i_hbm.at[pl.ds(base, BPW)], i_v)
        rows_lo = lax.iota(jnp.int32, GROUP)
        rows_hi = rows_lo + GROUP

        def step(g, _):
            uvec = u_v[pl.ds(g * GROUP, GROUP)]
            ivec = i_v[pl.ds(g * GROUP, GROUP)]
            us = [uvec[k_] for k_ in range(GROUP)]
            is_ = [ivec[k_] for k_ in range(GROUP)]

            def fire_one(k_):
                s = k_ % RING
                utc = pl.multiple_of((us[k_] >> 7) << 7, 128)
                itc = pl.multiple_of((is_[k_] >> 7) << 7, 128)
                return [
                    pltpu.async_copy(ug_t.at[:, pl.ds(utc, 128)],
                                     rings[0 * RING + s], sems[s]),
                    pltpu.async_copy(um_t.at[:, pl.ds(utc, 128)],
                                     rings[1 * RING + s], sems[s]),
                    pltpu.async_copy(ig_t.at[:, pl.ds(itc, 128)],
                                     rings[2 * RING + s], sems[s]),
                    pltpu.async_copy(im_t.at[:, pl.ds(itc, 128)],
                                     rings[3 * RING + s], sems[s]),
                ]

            def extract_one(k_, cps):
                s = k_ % RING
                for c in cps:
                    c.wait()
                slot = g * GROUP + k_
                cols_s = jnp.full((GROUP,), slot, jnp.int32)
                uc = jnp.full((GROUP,), us[k_] & 127, jnp.int32)
                ic = jnp.full((GROUP,), is_[k_] & 127, jnp.int32)
                for t, (out, cvec) in enumerate(
                        zip(outs, (uc, uc, ic, ic))):
                    buf = rings[t * RING + s]
                    v_lo = plsc.load_gather(buf, [rows_lo, cvec])
                    v_hi = plsc.load_gather(buf, [rows_hi, cvec])
                    plsc.store_scatter(out, [rows_lo, cols_s], v_lo)
                    plsc.store_scatter(out, [rows_hi, cols_s], v_hi)

            pending = {}
            for k_ in range(RING - 1):
                pending[k_] = fire_one(k_)
            for k_ in range(GROUP):
                if k_ + RING - 1 < GROUP:
                    pending[k_ + RING - 1] = fire_one(k_ + RING - 1)
                extract_one(k_, pending.pop(k_))
            return 0

        lax.fori_loop(0, BPW // GROUP, step, 0)
        dst = pl.ds(base, BPW)
        pltpu.sync_copy(bug, ug_o.at[:, dst])
        pltpu.sync_copy(bum, um_o.at[:, dst])
        pltpu.sync_copy(big, ig_o.at[:, dst])
        pltpu.sync_copy(bim, im_o.at[:, dst])

    return k(u, i, ugT, igT, umT, imT)


BLK = 8192


def _tc_mlp(ugT, igT, umT, imT, w1t, b1, w2t, b2, w3t, b3, wg, wx, bout):
    def body(ug_ref, ig_ref, um_ref, im_ref, w1_ref, b1_ref, w2_ref, b2_ref,
             w3_ref, b3_ref, wg_ref, wx_ref, bo_ref, o_ref):
        gmf = ug_ref[...] * ig_ref[...]
        x = jnp.concatenate([um_ref[...], im_ref[...]], axis=0)
        x = jnp.maximum(
            jnp.dot(w1_ref[...], x, preferred_element_type=jnp.float32)
            + b1_ref[...], 0.0)
        x = jnp.maximum(
            jnp.dot(w2_ref[...], x, preferred_element_type=jnp.float32)
            + b2_ref[...], 0.0)
        x = jnp.maximum(
            jnp.dot(w3_ref[...], x, preferred_element_type=jnp.float32)
            + b3_ref[...], 0.0)
        o = (jnp.dot(wg_ref[...], gmf, preferred_element_type=jnp.float32)
             + jnp.dot(wx_ref[...], x, preferred_element_type=jnp.float32)
             + bo_ref[...])
        o_ref[...] = 1.0 / (1.0 + jnp.exp(-o))

    blk_spec = pl.BlockSpec((D, BLK), lambda n: (0, n))
    full = lambda s: pl.BlockSpec(s, lambda n: tuple(0 for _ in s))
    return pl.pallas_call(
        body,
        grid=(B // BLK,),
        in_specs=[
            blk_spec, blk_spec, blk_spec, blk_spec,
            full((128, 64)), full((128, 1)),
            full((64, 128)), full((64, 1)),
            full((32, 64)), full((32, 1)),
            full((1, 32)), full((1, 32)), full((1, 1)),
        ],
        out_specs=pl.BlockSpec((1, BLK), lambda n: (0, n)),
        out_shape=jax.ShapeDtypeStruct((1, B), jnp.float32),
    )(ugT, igT, umT, imT, w1t, b1, w2t, b2, w3t, b3, wg, wx, bout)


def kernel(u, i, user_gmf, item_gmf, user_mlp, item_mlp,
           W1, b1, W2, b2, W3, b3, Wout, bout):
    ugT, igT = jnp.transpose(user_gmf), jnp.transpose(item_gmf)
    umT, imT = jnp.transpose(user_mlp), jnp.transpose(item_mlp)
    g_ugT, g_igT, g_umT, g_imT = _sc_gather(u, i, ugT, igT, umT, imT)
    out = _tc_mlp(g_ugT, g_igT, g_umT, g_imT,
                  W1.T, b1.reshape(-1, 1), W2.T, b2.reshape(-1, 1),
                  W3.T, b3.reshape(-1, 1),
                  Wout[:D, 0].reshape(1, D), Wout[D:, 0].reshape(1, D),
                  bout.reshape(1, 1))
    return out.reshape(B)
